# Initial kernel scaffold; baseline (speedup 1.0000x reference)
import functools

import jax
import jax.numpy as jnp
from jax import lax
from jax.experimental import pallas as pl
from jax.experimental.pallas import tpu as pltpu
from jax.experimental.pallas import tpu_sc as plsc

N_NODES = 10000
N_EDGES = 320000
D = 128
DEG_W = 16

NC = 2
NS = 16
NW = NC * NS
E_PER_TILE = N_EDGES // NW
K = 80
N_BATCH = E_PER_TILE // K


def _sc_aggregate(x, src, dst, zero_aggr, zero_deg, ones_rows):
    mesh = plsc.VectorSubcoreMesh(core_axis_name="c", subcore_axis_name="s")

    @functools.partial(
        pl.kernel,
        mesh=mesh,
        out_type=[
            jax.ShapeDtypeStruct((NC, N_NODES, D), jnp.float32),
            jax.ShapeDtypeStruct((NC, N_NODES, DEG_W), jnp.float32),
        ],
        scratch_types=[
            pltpu.VMEM((K,), jnp.int32),
            pltpu.VMEM((K,), jnp.int32),
            pltpu.VMEM((K, D), jnp.float32),
            pltpu.VMEM((K, DEG_W), jnp.float32),
            pltpu.VMEM_SHARED((N_NODES, D), jnp.float32),
            pltpu.VMEM_SHARED((N_NODES, DEG_W), jnp.float32),
            pltpu.SemaphoreType.DMA,
        ],
    )
    def k(x_hbm, src_hbm, dst_hbm, zag_hbm, zdg_hbm, ones_hbm,
          aggr_out, deg_out,
          src_v, dst_v, rows_v, ones_v, aggr_sh, deg_sh, sem):
        c = lax.axis_index("c")
        s = lax.axis_index("s")
        wid = s * NC + c

        @pl.when(s == 0)
        def _init():
            pltpu.sync_copy(zag_hbm, aggr_sh)
            pltpu.sync_copy(zdg_hbm, deg_sh)

        pltpu.sync_copy(ones_hbm, ones_v)
        plsc.subcore_barrier()

        base = wid * E_PER_TILE

        def body(b, carry):
            off = base + b * K
            pltpu.sync_copy(src_hbm.at[pl.ds(off, K)], src_v)
            pltpu.sync_copy(dst_hbm.at[pl.ds(off, K)], dst_v)
            pltpu.async_copy(x_hbm.at[src_v], rows_v, sem).wait()
            pltpu.sync_copy(rows_v, aggr_sh.at[dst_v], add=True)
            pltpu.sync_copy(ones_v, deg_sh.at[dst_v], add=True)
            return carry

        lax.fori_loop(0, N_BATCH, body, 0)
        plsc.subcore_barrier()

        @pl.when(s == 0)
        def _flush():
            pltpu.sync_copy(aggr_sh, aggr_out.at[c])
            pltpu.sync_copy(deg_sh, deg_out.at[c])

    return k(x, src, dst, zero_aggr, zero_deg, ones_rows)


BLK = 400


def _tc_combine(p, d2, x, W_l, b_l, W_r):
    def body(p_ref, d_ref, x_ref, wl_ref, bl_ref, wr_ref, o_ref):
        ssum = p_ref[0] + p_ref[1]
        deg = d_ref[0, :, 0:1] + d_ref[1, :, 0:1]
        deg = jnp.maximum(deg, 1.0)
        aggr = ssum / deg
        acc = jnp.dot(aggr, wl_ref[...], preferred_element_type=jnp.float32)
        acc = acc + jnp.dot(x_ref[...], wr_ref[...],
                            preferred_element_type=jnp.float32)
        acc = acc + bl_ref[...]
        o_ref[...] = jnp.maximum(acc, 0.0)

    return pl.pallas_call(
        body,
        grid=(N_NODES // BLK,),
        in_specs=[
            pl.BlockSpec((NC, BLK, D), lambda i: (0, i, 0)),
            pl.BlockSpec((NC, BLK, DEG_W), lambda i: (0, i, 0)),
            pl.BlockSpec((BLK, D), lambda i: (i, 0)),
            pl.BlockSpec((D, D), lambda i: (0, 0)),
            pl.BlockSpec((1, D), lambda i: (0, 0)),
            pl.BlockSpec((D, D), lambda i: (0, 0)),
        ],
        out_specs=pl.BlockSpec((BLK, D), lambda i: (i, 0)),
        out_shape=jax.ShapeDtypeStruct((N_NODES, D), jnp.float32),
    )(p, d2, x, W_l, b_l.reshape(1, D), W_r)


def kernel(x, edge_index, W_l, b_l, W_r):
    src = edge_index[0].astype(jnp.int32)
    dst = edge_index[1].astype(jnp.int32)
    zero_aggr = jnp.zeros((N_NODES, D), jnp.float32)
    zero_deg = jnp.zeros((N_NODES, DEG_W), jnp.float32)
    ones_rows = jnp.ones((K, DEG_W), jnp.float32)
    p, d2 = _sc_aggregate(x, src, dst, zero_aggr, zero_deg, ones_rows)
    return _tc_combine(p, d2, x, W_l, b_l, W_r)


# speedup vs baseline: 5.5217x; 5.5217x over previous
import functools

import jax
import jax.numpy as jnp
from jax import lax
from jax.experimental import pallas as pl
from jax.experimental.pallas import tpu as pltpu
from jax.experimental.pallas import tpu_sc as plsc

N_NODES = 10000
N_PAD = 10240
N_EDGES = 320000
D = 128
DW = 144
RPT = N_PAD // 16

NC = 2
NS = 16
NW = NC * NS
E_PER_TILE = N_EDGES // NW
K = 80
N_BATCH = E_PER_TILE // K
RCHUNK = 32


def _sc_aggregate(xe, src, dst, zero_aggr):
    mesh = plsc.VectorSubcoreMesh(core_axis_name="c", subcore_axis_name="s")

    @functools.partial(
        pl.kernel,
        mesh=mesh,
        out_type=jax.ShapeDtypeStruct((NC, N_PAD, DW), jnp.float32),
        compiler_params=pltpu.CompilerParams(use_tc_tiling_on_sc=False),
        scratch_types=[
            pltpu.VMEM((K,), jnp.int32),
            pltpu.VMEM((K,), jnp.int32),
            pltpu.VMEM((K, DW), jnp.float32),
            pltpu.VMEM((RCHUNK, DW), jnp.float32),
            pltpu.VMEM_SHARED((N_PAD, DW), jnp.float32),
            pltpu.SemaphoreType.DMA,
        ],
    )
    def k(xe_hbm, src_hbm, dst_hbm, zag_hbm,
          aggr_out,
          src_v, dst_v, rows_v, stg_v, aggr_sh, sem):
        c = lax.axis_index("c")
        s = lax.axis_index("s")
        wid = s * NC + c

        pltpu.sync_copy(zag_hbm, stg_v)

        for j in range(RPT // RCHUNK):
            r0 = s * RPT + j * RCHUNK
            pltpu.sync_copy(stg_v, aggr_sh.at[pl.ds(r0, RCHUNK)])

        plsc.subcore_barrier()

        base = wid * E_PER_TILE

        def body(b, carry):
            off = base + b * K
            pltpu.sync_copy(src_hbm.at[pl.ds(off, K)], src_v)
            pltpu.sync_copy(dst_hbm.at[pl.ds(off, K)], dst_v)
            pltpu.async_copy(xe_hbm.at[src_v], rows_v, sem).wait()
            pltpu.sync_copy(rows_v, aggr_sh.at[dst_v], add=True)
            return carry

        lax.fori_loop(0, N_BATCH, body, 0)
        plsc.subcore_barrier()

        for j in range(RPT // RCHUNK):
            r0 = s * RPT + j * RCHUNK
            pltpu.sync_copy(aggr_sh.at[pl.ds(r0, RCHUNK)], stg_v)
            pltpu.sync_copy(stg_v, aggr_out.at[c, pl.ds(r0, RCHUNK)])

    return k(xe, src, dst, zero_aggr)


BLK = 400


def _tc_combine(p, x, W_l, b_l, W_r):
    def body(p_ref, x_ref, wl_ref, bl_ref, wr_ref, o_ref):
        ssum = p_ref[0] + p_ref[1]
        deg = jnp.maximum(ssum[:, D:D + 1], 1.0)
        aggr = ssum[:, 0:D] / deg
        acc = jnp.dot(aggr, wl_ref[...], preferred_element_type=jnp.float32)
        acc = acc + jnp.dot(x_ref[...], wr_ref[...],
                            preferred_element_type=jnp.float32)
        acc = acc + bl_ref[...]
        o_ref[...] = jnp.maximum(acc, 0.0)

    return pl.pallas_call(
        body,
        grid=(N_NODES // BLK,),
        in_specs=[
            pl.BlockSpec((NC, BLK, DW), lambda i: (0, i, 0)),
            pl.BlockSpec((BLK, D), lambda i: (i, 0)),
            pl.BlockSpec((D, D), lambda i: (0, 0)),
            pl.BlockSpec((1, D), lambda i: (0, 0)),
            pl.BlockSpec((D, D), lambda i: (0, 0)),
        ],
        out_specs=pl.BlockSpec((BLK, D), lambda i: (i, 0)),
        out_shape=jax.ShapeDtypeStruct((N_NODES, D), jnp.float32),
    )(p, x, W_l, b_l.reshape(1, D), W_r)


def kernel(x, edge_index, W_l, b_l, W_r):
    src = edge_index[0].astype(jnp.int32)
    dst = edge_index[1].astype(jnp.int32)
    xe = jnp.concatenate(
        [x, jnp.ones((N_NODES, 1), jnp.float32),
         jnp.zeros((N_NODES, DW - D - 1), jnp.float32)], axis=1)
    zero_aggr = jnp.zeros((RCHUNK, DW), jnp.float32)
    p = _sc_aggregate(xe, src, dst, zero_aggr)
    return _tc_combine(p, x, W_l, b_l, W_r)


# speedup vs baseline: 6.3433x; 1.1488x over previous
import functools

import jax
import jax.numpy as jnp
from jax import lax
from jax.experimental import pallas as pl
from jax.experimental.pallas import tpu as pltpu
from jax.experimental.pallas import tpu_sc as plsc

N_NODES = 10000
N_PAD = 10112
N_EDGES = 320000
D = 128
DW = 144
RPT = N_PAD // 16

NC = 2
NS = 16
NW = NC * NS
E_PER_TILE = N_EDGES // NW
K = 48
NPAIR = 104
TAIL = 16


def _sc_aggregate(xe, em, et, zeros48):
    mesh = plsc.VectorSubcoreMesh(core_axis_name="c", subcore_axis_name="s")

    @functools.partial(
        pl.kernel,
        mesh=mesh,
        out_type=jax.ShapeDtypeStruct((NC, N_PAD, DW), jnp.float32),
        compiler_params=pltpu.CompilerParams(use_tc_tiling_on_sc=False),
        scratch_types=[
            pltpu.VMEM((2, 2, K), jnp.int32),
            pltpu.VMEM((2, 2, K), jnp.int32),
            pltpu.VMEM((2, TAIL), jnp.int32),
            pltpu.VMEM((K, DW), jnp.float32),
            pltpu.VMEM((K, DW), jnp.float32),
            pltpu.VMEM((TAIL, DW), jnp.float32),
            pltpu.VMEM_SHARED((N_PAD, DW), jnp.float32),
            pltpu.SemaphoreType.DMA,
            pltpu.SemaphoreType.DMA,
            pltpu.SemaphoreType.DMA,
        ],
    )
    def k(xe_hbm, em_hbm, et_hbm, z_hbm,
          aggr_out,
          eidxA, eidxB, tidx, rowsA, rowsB, rowsT, aggr_sh,
          semA, semB, semI):
        c = lax.axis_index("c")
        s = lax.axis_index("s")
        wid = s * NC + c

        pltpu.sync_copy(z_hbm, rowsA)
        for j in range(13):
            pltpu.sync_copy(rowsA, aggr_sh.at[pl.ds(s * RPT + j * K, K)])
        pltpu.sync_copy(rowsA.at[pl.ds(0, 8)],
                        aggr_sh.at[pl.ds(s * RPT + 13 * K, 8)])
        plsc.subcore_barrier()

        def pair_step(p, cur_idx, nxt_idx, cur_rows, oth_rows,
                      cur_sem, oth_sem):
            pf = pltpu.async_copy(em_hbm.at[wid, p + 1], nxt_idx, semI)
            pltpu.make_async_copy(
                xe_hbm.at[cur_idx.at[0, 0]], cur_rows, cur_sem).wait()
            g2 = pltpu.async_copy(
                xe_hbm.at[cur_idx.at[0, 1]], oth_rows, oth_sem)
            pltpu.sync_copy(cur_rows, aggr_sh.at[cur_idx.at[1, 0]], add=True)
            pf.wait()
            pltpu.async_copy(
                xe_hbm.at[nxt_idx.at[0, 0]], cur_rows, cur_sem)
            g2.wait()
            pltpu.sync_copy(oth_rows, aggr_sh.at[cur_idx.at[1, 1]], add=True)

        pltpu.sync_copy(em_hbm.at[wid, 0], eidxA)
        pltpu.async_copy(xe_hbm.at[eidxA.at[0, 0]], rowsA, semA)

        def body(j, carry):
            pair_step(2 * j, eidxA, eidxB, rowsA, rowsB, semA, semB)
            pair_step(2 * j + 1, eidxB, eidxA, rowsA, rowsB, semA, semB)
            return carry

        lax.fori_loop(0, NPAIR // 2, body, 0)

        pltpu.make_async_copy(xe_hbm.at[eidxA.at[0, 0]], rowsA, semA).wait()

        pltpu.sync_copy(et_hbm.at[wid], tidx)
        pltpu.async_copy(xe_hbm.at[tidx.at[0]], rowsT, semB).wait()
        pltpu.sync_copy(rowsT, aggr_sh.at[tidx.at[1]], add=True)

        plsc.subcore_barrier()

        for j in range(13):
            r0 = s * RPT + j * K
            pltpu.sync_copy(aggr_sh.at[pl.ds(r0, K)], rowsA)
            pltpu.sync_copy(rowsA, aggr_out.at[c, pl.ds(r0, K)])
        r0 = s * RPT + 13 * K
        pltpu.sync_copy(aggr_sh.at[pl.ds(r0, 8)], rowsT.at[pl.ds(0, 8)])
        pltpu.sync_copy(rowsT.at[pl.ds(0, 8)], aggr_out.at[c, pl.ds(r0, 8)])

    return k(xe, em, et, zeros48)


BLK = 400


def _tc_combine(p, x, W_l, b_l, W_r):
    def body(p_ref, x_ref, wl_ref, bl_ref, wr_ref, o_ref):
        ssum = p_ref[0] + p_ref[1]
        deg = jnp.maximum(ssum[:, D:D + 1], 1.0)
        aggr = ssum[:, 0:D] / deg
        acc = jnp.dot(aggr, wl_ref[...], preferred_element_type=jnp.float32)
        acc = acc + jnp.dot(x_ref[...], wr_ref[...],
                            preferred_element_type=jnp.float32)
        acc = acc + bl_ref[...]
        o_ref[...] = jnp.maximum(acc, 0.0)

    return pl.pallas_call(
        body,
        grid=(N_NODES // BLK,),
        in_specs=[
            pl.BlockSpec((NC, BLK, DW), lambda i: (0, i, 0)),
            pl.BlockSpec((BLK, D), lambda i: (i, 0)),
            pl.BlockSpec((D, D), lambda i: (0, 0)),
            pl.BlockSpec((1, D), lambda i: (0, 0)),
            pl.BlockSpec((D, D), lambda i: (0, 0)),
        ],
        out_specs=pl.BlockSpec((BLK, D), lambda i: (i, 0)),
        out_shape=jax.ShapeDtypeStruct((N_NODES, D), jnp.float32),
    )(p, x, W_l, b_l.reshape(1, D), W_r)


def kernel(x, edge_index, W_l, b_l, W_r):
    src = edge_index[0].astype(jnp.int32).reshape(NW, E_PER_TILE)
    dst = edge_index[1].astype(jnp.int32).reshape(NW, E_PER_TILE)
    main = NPAIR * 2 * K
    srcm = src[:, :main].reshape(NW, NPAIR, 2, K)
    dstm = dst[:, :main].reshape(NW, NPAIR, 2, K)
    em = jnp.stack([srcm, dstm], axis=2)
    em = jnp.pad(em, ((0, 0), (0, 1), (0, 0), (0, 0), (0, 0)))
    et = jnp.stack([src[:, main:], dst[:, main:]], axis=1)
    xe = jnp.concatenate(
        [x, jnp.ones((N_NODES, 1), jnp.float32),
         jnp.zeros((N_NODES, DW - D - 1), jnp.float32)], axis=1)
    zeros48 = jnp.zeros((K, DW), jnp.float32)
    p = _sc_aggregate(xe, em, et, zeros48)
    return _tc_combine(p, x, W_l, b_l, W_r)


# speedup vs baseline: 7.0173x; 1.1063x over previous
import functools

import jax
import jax.numpy as jnp
from jax import lax
from jax.experimental import pallas as pl
from jax.experimental.pallas import tpu as pltpu
from jax.experimental.pallas import tpu_sc as plsc

N_NODES = 10000
N_PAD = 10112
N_EDGES = 320000
D = 128
RPT = N_PAD // 16

NC = 2
NS = 16
NW = NC * NS
E_PER_TILE = N_EDGES // NW
K = 48
NPAIR = 104
TAIL = 16


def _sc_aggregate(x, em, et, zeros48):
    mesh = plsc.VectorSubcoreMesh(core_axis_name="c", subcore_axis_name="s")

    @functools.partial(
        pl.kernel,
        mesh=mesh,
        out_type=[
            jax.ShapeDtypeStruct((NC, N_PAD, D), jnp.float32),
            jax.ShapeDtypeStruct((NC, NS, N_PAD), jnp.float32),
        ],
        compiler_params=pltpu.CompilerParams(use_tc_tiling_on_sc=False,
                                             needs_layout_passes=False),
        scratch_types=[
            pltpu.VMEM((2, 2, K), jnp.int32),
            pltpu.VMEM((2, 2, K), jnp.int32),
            pltpu.VMEM((2, TAIL), jnp.int32),
            pltpu.VMEM((K, D), jnp.float32),
            pltpu.VMEM((K, D), jnp.float32),
            pltpu.VMEM((TAIL, D), jnp.float32),
            pltpu.VMEM((N_PAD,), jnp.float32),
            pltpu.VMEM_SHARED((N_PAD, D), jnp.float32),
            pltpu.SemaphoreType.DMA,
            pltpu.SemaphoreType.DMA,
            pltpu.SemaphoreType.DMA,
        ],
    )
    def k(x_hbm, em_hbm, et_hbm, z_hbm,
          aggr_out, hist_out,
          eidxA, eidxB, tidx, rowsA, rowsB, rowsT, hist_v, aggr_sh,
          semA, semB, semI):
        c = lax.axis_index("c")
        s = lax.axis_index("s")
        wid = s * NC + c

        zeros16 = jnp.zeros((16,), jnp.float32)
        ones16 = jnp.ones((16,), jnp.float32)

        def zh(i, carry):
            hist_v[pl.ds(i * 16, 16)] = zeros16
            return carry

        lax.fori_loop(0, N_PAD // 16, zh, 0)

        pltpu.sync_copy(z_hbm, rowsA)
        for j in range(13):
            pltpu.sync_copy(rowsA, aggr_sh.at[pl.ds(s * RPT + j * K, K)])
        pltpu.sync_copy(rowsA.at[pl.ds(0, 8)],
                        aggr_sh.at[pl.ds(s * RPT + 13 * K, 8)])
        plsc.subcore_barrier()

        def histo(idx_ref, a, b, n):
            for g in range(n // 16):
                dvec = idx_ref[a, b, pl.ds(g * 16, 16)]
                plsc.addupdate_scatter(hist_v, [dvec], ones16)

        def pair_step(p, cur_idx, nxt_idx):
            pf = pltpu.async_copy(em_hbm.at[wid, p + 1], nxt_idx, semI)
            pltpu.make_async_copy(
                x_hbm.at[cur_idx.at[0, 0]], rowsA, semA).wait()
            g2 = pltpu.async_copy(
                x_hbm.at[cur_idx.at[0, 1]], rowsB, semB)
            pltpu.sync_copy(rowsA, aggr_sh.at[cur_idx.at[1, 0]], add=True)
            histo(cur_idx, 1, 0, K)
            pf.wait()
            pltpu.async_copy(
                x_hbm.at[nxt_idx.at[0, 0]], rowsA, semA)
            g2.wait()
            pltpu.sync_copy(rowsB, aggr_sh.at[cur_idx.at[1, 1]], add=True)
            histo(cur_idx, 1, 1, K)

        pltpu.sync_copy(em_hbm.at[wid, 0], eidxA)
        pltpu.async_copy(x_hbm.at[eidxA.at[0, 0]], rowsA, semA)

        def body(j, carry):
            pair_step(2 * j, eidxA, eidxB)
            pair_step(2 * j + 1, eidxB, eidxA)
            return carry

        lax.fori_loop(0, NPAIR // 2, body, 0)

        pltpu.make_async_copy(x_hbm.at[eidxA.at[0, 0]], rowsA, semA).wait()

        pltpu.sync_copy(et_hbm.at[wid], tidx)
        pltpu.async_copy(x_hbm.at[tidx.at[0]], rowsT, semB).wait()
        pltpu.sync_copy(rowsT, aggr_sh.at[tidx.at[1]], add=True)
        dvec_t = tidx[1, pl.ds(0, 16)]
        plsc.addupdate_scatter(hist_v, [dvec_t], ones16)

        pltpu.sync_copy(hist_v, hist_out.at[c, s])

        plsc.subcore_barrier()

        for j in range(13):
            r0 = s * RPT + j * K
            pltpu.sync_copy(aggr_sh.at[pl.ds(r0, K)], rowsA)
            pltpu.sync_copy(rowsA, aggr_out.at[c, pl.ds(r0, K)])
        r0 = s * RPT + 13 * K
        pltpu.sync_copy(aggr_sh.at[pl.ds(r0, 8)], rowsT.at[pl.ds(0, 8)])
        pltpu.sync_copy(rowsT.at[pl.ds(0, 8)], aggr_out.at[c, pl.ds(r0, 8)])

    return k(x, em, et, zeros48)


BLK = 400


def _tc_combine(p, hist, x, W_l, b_l, W_r):
    def body(p_ref, h_ref, x_ref, wl_ref, bl_ref, wr_ref, o_ref):
        ssum = p_ref[0] + p_ref[1]
        deg = jnp.sum(h_ref[...], axis=1)[:, None]
        deg = jnp.maximum(deg, 1.0)
        aggr = ssum / deg
        acc = jnp.dot(aggr, wl_ref[...], preferred_element_type=jnp.float32)
        acc = acc + jnp.dot(x_ref[...], wr_ref[...],
                            preferred_element_type=jnp.float32)
        acc = acc + bl_ref[...]
        o_ref[...] = jnp.maximum(acc, 0.0)

    return pl.pallas_call(
        body,
        grid=(N_NODES // BLK,),
        in_specs=[
            pl.BlockSpec((NC, BLK, D), lambda i: (0, i, 0)),
            pl.BlockSpec((BLK, NC * NS), lambda i: (i, 0)),
            pl.BlockSpec((BLK, D), lambda i: (i, 0)),
            pl.BlockSpec((D, D), lambda i: (0, 0)),
            pl.BlockSpec((1, D), lambda i: (0, 0)),
            pl.BlockSpec((D, D), lambda i: (0, 0)),
        ],
        out_specs=pl.BlockSpec((BLK, D), lambda i: (i, 0)),
        out_shape=jax.ShapeDtypeStruct((N_NODES, D), jnp.float32),
    )(p, hist.reshape(NC * NS, N_PAD).T, x, W_l, b_l.reshape(1, D), W_r)


def kernel(x, edge_index, W_l, b_l, W_r):
    src = edge_index[0].astype(jnp.int32).reshape(NW, E_PER_TILE)
    dst = edge_index[1].astype(jnp.int32).reshape(NW, E_PER_TILE)
    main = NPAIR * 2 * K
    srcm = src[:, :main].reshape(NW, NPAIR, 2, K)
    dstm = dst[:, :main].reshape(NW, NPAIR, 2, K)
    em = jnp.stack([srcm, dstm], axis=2)
    em = jnp.pad(em, ((0, 0), (0, 1), (0, 0), (0, 0), (0, 0)))
    et = jnp.stack([src[:, main:], dst[:, main:]], axis=1)
    zeros48 = jnp.zeros((K, D), jnp.float32)
    p, hist = _sc_aggregate(x, em, et, zeros48)
    return _tc_combine(p, hist, x, W_l, b_l, W_r)


# speedup vs baseline: 7.9658x; 1.1352x over previous
import functools

import jax
import jax.numpy as jnp
from jax import lax
from jax.experimental import pallas as pl
from jax.experimental.pallas import tpu as pltpu
from jax.experimental.pallas import tpu_sc as plsc

N_NODES = 10000
N_PAD = 10112
N_EDGES = 320000
D = 128
RPT = N_PAD // 16

NC = 2
NS = 16
NW = NC * NS
E_PER_TILE = N_EDGES // NW
K = 64
NPAIR = 78
TAIL = 16


def _sc_aggregate(x, em, et, zeros48):
    mesh = plsc.VectorSubcoreMesh(core_axis_name="c", subcore_axis_name="s")

    @functools.partial(
        pl.kernel,
        mesh=mesh,
        out_type=[
            jax.ShapeDtypeStruct((NC, N_PAD, D), jnp.float32),
            jax.ShapeDtypeStruct((NC, NS, N_PAD), jnp.float32),
        ],
        compiler_params=pltpu.CompilerParams(use_tc_tiling_on_sc=False,
                                             needs_layout_passes=False),
        scratch_types=[
            pltpu.VMEM((2, 2, K), jnp.int32),
            pltpu.VMEM((2, 2, K), jnp.int32),
            pltpu.VMEM((2, TAIL), jnp.int32),
            pltpu.VMEM((K, D), jnp.float32),
            pltpu.VMEM((K, D), jnp.float32),
            pltpu.VMEM((TAIL, D), jnp.float32),
            pltpu.VMEM((N_PAD,), jnp.float32),
            pltpu.VMEM_SHARED((N_PAD, D), jnp.float32),
            pltpu.SemaphoreType.DMA,
            pltpu.SemaphoreType.DMA,
            pltpu.SemaphoreType.DMA,
        ],
    )
    def k(x_hbm, em_hbm, et_hbm, z_hbm,
          aggr_out, hist_out,
          eidxA, eidxB, tidx, rowsA, rowsB, rowsT, hist_v, aggr_sh,
          semA, semB, semI):
        c = lax.axis_index("c")
        s = lax.axis_index("s")
        wid = s * NC + c

        zeros16 = jnp.zeros((16,), jnp.float32)
        ones16 = jnp.ones((16,), jnp.float32)

        def zh(i, carry):
            hist_v[pl.ds(i * 16, 16)] = zeros16
            return carry

        lax.fori_loop(0, N_PAD // 16, zh, 0)

        pltpu.sync_copy(z_hbm, rowsA)
        for j in range(9):
            pltpu.sync_copy(rowsA, aggr_sh.at[pl.ds(s * RPT + j * K, K)])
        pltpu.sync_copy(rowsA.at[pl.ds(0, 56)],
                        aggr_sh.at[pl.ds(s * RPT + 9 * K, 56)])
        plsc.subcore_barrier()

        def histo(idx_ref, a, b, n):
            for g in range(n // 16):
                dvec = idx_ref[a, b, pl.ds(g * 16, 16)]
                plsc.addupdate_scatter(hist_v, [dvec], ones16)

        def pair_step(p, cur_idx, nxt_idx):
            pf = pltpu.async_copy(em_hbm.at[wid, p + 1], nxt_idx, semI)
            pltpu.make_async_copy(
                x_hbm.at[cur_idx.at[0, 0]], rowsA, semA).wait()
            g2 = pltpu.async_copy(
                x_hbm.at[cur_idx.at[0, 1]], rowsB, semB)
            pltpu.sync_copy(rowsA, aggr_sh.at[cur_idx.at[1, 0]], add=True)
            histo(cur_idx, 1, 0, K)
            pf.wait()
            pltpu.async_copy(
                x_hbm.at[nxt_idx.at[0, 0]], rowsA, semA)
            g2.wait()
            pltpu.sync_copy(rowsB, aggr_sh.at[cur_idx.at[1, 1]], add=True)
            histo(cur_idx, 1, 1, K)

        pltpu.sync_copy(em_hbm.at[wid, 0], eidxA)
        pltpu.async_copy(x_hbm.at[eidxA.at[0, 0]], rowsA, semA)

        def body(j, carry):
            pair_step(2 * j, eidxA, eidxB)
            pair_step(2 * j + 1, eidxB, eidxA)
            return carry

        lax.fori_loop(0, NPAIR // 2, body, 0)

        pltpu.make_async_copy(x_hbm.at[eidxA.at[0, 0]], rowsA, semA).wait()

        pltpu.sync_copy(et_hbm.at[wid], tidx)
        pltpu.async_copy(x_hbm.at[tidx.at[0]], rowsT, semB).wait()
        pltpu.sync_copy(rowsT, aggr_sh.at[tidx.at[1]], add=True)
        dvec_t = tidx[1, pl.ds(0, 16)]
        plsc.addupdate_scatter(hist_v, [dvec_t], ones16)

        pltpu.sync_copy(hist_v, hist_out.at[c, s])

        plsc.subcore_barrier()

        for j in range(9):
            r0 = s * RPT + j * K
            pltpu.sync_copy(aggr_sh.at[pl.ds(r0, K)], rowsA)
            pltpu.sync_copy(rowsA, aggr_out.at[c, pl.ds(r0, K)])
        r0 = s * RPT + 9 * K
        pltpu.sync_copy(aggr_sh.at[pl.ds(r0, 56)], rowsA.at[pl.ds(0, 56)])
        pltpu.sync_copy(rowsA.at[pl.ds(0, 56)], aggr_out.at[c, pl.ds(r0, 56)])

    return k(x, em, et, zeros48)


BLK = 400


def _tc_combine(p, hist, x, W_l, b_l, W_r):
    def body(p_ref, h_ref, x_ref, wl_ref, bl_ref, wr_ref, o_ref):
        ssum = p_ref[0] + p_ref[1]
        deg = jnp.sum(h_ref[...], axis=1)[:, None]
        deg = jnp.maximum(deg, 1.0)
        aggr = ssum / deg
        acc = jnp.dot(aggr, wl_ref[...], preferred_element_type=jnp.float32)
        acc = acc + jnp.dot(x_ref[...], wr_ref[...],
                            preferred_element_type=jnp.float32)
        acc = acc + bl_ref[...]
        o_ref[...] = jnp.maximum(acc, 0.0)

    return pl.pallas_call(
        body,
        grid=(N_NODES // BLK,),
        in_specs=[
            pl.BlockSpec((NC, BLK, D), lambda i: (0, i, 0)),
            pl.BlockSpec((BLK, NC * NS), lambda i: (i, 0)),
            pl.BlockSpec((BLK, D), lambda i: (i, 0)),
            pl.BlockSpec((D, D), lambda i: (0, 0)),
            pl.BlockSpec((1, D), lambda i: (0, 0)),
            pl.BlockSpec((D, D), lambda i: (0, 0)),
        ],
        out_specs=pl.BlockSpec((BLK, D), lambda i: (i, 0)),
        out_shape=jax.ShapeDtypeStruct((N_NODES, D), jnp.float32),
    )(p, hist.reshape(NC * NS, N_PAD).T, x, W_l, b_l.reshape(1, D), W_r)


def kernel(x, edge_index, W_l, b_l, W_r):
    src = edge_index[0].astype(jnp.int32).reshape(NW, E_PER_TILE)
    dst = edge_index[1].astype(jnp.int32).reshape(NW, E_PER_TILE)
    main = NPAIR * 2 * K
    srcm = src[:, :main].reshape(NW, NPAIR, 2, K)
    dstm = dst[:, :main].reshape(NW, NPAIR, 2, K)
    em = jnp.stack([srcm, dstm], axis=2)
    em = jnp.pad(em, ((0, 0), (0, 1), (0, 0), (0, 0), (0, 0)))
    et = jnp.stack([src[:, main:], dst[:, main:]], axis=1)
    zeros48 = jnp.zeros((K, D), jnp.float32)
    p, hist = _sc_aggregate(x, em, et, zeros48)
    return _tc_combine(p, hist, x, W_l, b_l, W_r)
